# baseline (speedup 1.0000x reference)
import functools

import jax
import jax.numpy as jnp
from jax import lax
from jax.experimental import pallas as pl
from jax.experimental.pallas import tpu as pltpu
from jax.experimental.pallas import tpu_sc as plsc

B, H, W = 16, 384, 384
P = 512
NC = 2
NS = 16
NW = NC * NS
PPT = (B * P) // NW
CH = 128
NCH = PPT // CH
LANES = 16

_mesh = plsc.VectorSubcoreMesh(
    core_axis_name="c", subcore_axis_name="s", num_cores=NC)


@functools.partial(
    pl.kernel,
    out_type=jax.ShapeDtypeStruct((NW, 2, LANES), jnp.float32),
    mesh=_mesh,
    scratch_types=[
        pltpu.VMEM((PPT,), jnp.float32),
        pltpu.VMEM((PPT,), jnp.float32),
        pltpu.VMEM((PPT,), jnp.float32),
        pltpu.VMEM((PPT,), jnp.int32),
        pltpu.VMEM((PPT,), jnp.float32),
        pltpu.VMEM((2, LANES), jnp.float32),
        pltpu.SemaphoreType.DMA,
        pltpu.SemaphoreType.DMA,
    ],
)
def _depth_partials(img_hbm, rcd_hbm, out_hbm,
                    rows_v, cols_v, depth_v, idx_v, vals_v, part_v,
                    in_sem, sem):
    wid = lax.axis_index("s") * NC + lax.axis_index("c")
    b = wid // 2
    off = (wid % 2) * PPT
    in_cps = [
        pltpu.async_copy(rcd_hbm.at[0, b, pl.ds(off, PPT)], rows_v, in_sem),
        pltpu.async_copy(rcd_hbm.at[1, b, pl.ds(off, PPT)], cols_v, in_sem),
        pltpu.async_copy(rcd_hbm.at[2, b, pl.ds(off, PPT)], depth_v, in_sem),
    ]
    for cp in in_cps:
        cp.wait()

    boff = b * (H // 8)
    copies = []
    for j in range(NCH):
        for k8 in range(CH // LANES):
            k = j * (CH // LANES) + k8
            r = rows_v[pl.ds(k * LANES, LANES)].astype(jnp.int32)
            c = cols_v[pl.ds(k * LANES, LANES)].astype(jnp.int32)
            tile = (boff + (r >> 3)) * 3 + (c >> 7)
            idx_v[pl.ds(k * LANES, LANES)] = (
                (tile << 10) + ((r & 7) << 7) + (c & 127))
        copies.append(pltpu.async_copy(
            img_hbm.at[idx_v.at[pl.ds(j * CH, CH)]],
            vals_v.at[pl.ds(j * CH, CH)], sem))
    for cp in copies:
        cp.wait()

    acc = jnp.zeros((LANES,), jnp.float32)
    cnt = jnp.zeros((LANES,), jnp.float32)
    for k in range(PPT // LANES):
        v = vals_v[pl.ds(k * LANES, LANES)]
        d = depth_v[pl.ds(k * LANES, LANES)]
        m = d > 0.0
        acc = acc + jnp.where(m, jnp.abs(v - d), 0.0)
        cnt = cnt + jnp.where(m, 1.0, 0.0)
    part_v[0, :] = acc
    part_v[1, :] = cnt
    pltpu.sync_copy(part_v, out_hbm.at[wid])


def _finalize_body(p_ref, o_ref):
    p = p_ref[...]
    s = jnp.sum(p[:, 0, :])
    c = jnp.sum(p[:, 1, :])
    loss = jnp.where(c > 0.0, s / jnp.maximum(c, 1.0), 0.0)
    o_ref[...] = jnp.broadcast_to(loss, (1, 1))


_finalize = pl.pallas_call(
    _finalize_body,
    out_shape=jax.ShapeDtypeStruct((1, 1), jnp.float32),
)


def kernel(output, rdepth):
    img = (output.reshape(B, H // 8, 8, W // 128, 128)
           .transpose(0, 1, 3, 2, 4)
           .reshape(-1))
    rcd = jnp.transpose(rdepth, (2, 0, 1))
    partials = _depth_partials(img, rcd)
    return _finalize(partials)[0, 0]


# speedup vs baseline: 1.0370x; 1.0370x over previous
import functools

import jax
import jax.numpy as jnp
from jax import lax
from jax.experimental import pallas as pl
from jax.experimental.pallas import tpu as pltpu
from jax.experimental.pallas import tpu_sc as plsc

B, H, W = 16, 384, 384
P = 512
NS = 16
PPT = (B * P) // NS
CH = 128
NCH = PPT // CH
LANES = 16

_mesh = plsc.VectorSubcoreMesh(
    core_axis_name="c", subcore_axis_name="s", num_cores=1)


@functools.partial(
    pl.kernel,
    out_type=jax.ShapeDtypeStruct((NS, 2, LANES), jnp.float32),
    mesh=_mesh,
    scratch_types=[
        pltpu.VMEM((PPT,), jnp.float32),
        pltpu.VMEM((PPT,), jnp.float32),
        pltpu.VMEM((PPT,), jnp.float32),
        pltpu.VMEM((PPT,), jnp.int32),
        pltpu.VMEM((PPT,), jnp.float32),
        pltpu.VMEM((2, LANES), jnp.float32),
        pltpu.SemaphoreType.DMA,
        pltpu.SemaphoreType.DMA,
        pltpu.SemaphoreType.DMA,
    ],
)
def _depth_partials(img_hbm, rcd_hbm, out_hbm,
                    rows_v, cols_v, depth_v, idx_v, vals_v, part_v,
                    rc_sem, d_sem, sem):
    sid = lax.axis_index("s")
    rc_cps = [
        pltpu.async_copy(rcd_hbm.at[0, sid], rows_v, rc_sem),
        pltpu.async_copy(rcd_hbm.at[1, sid], cols_v, rc_sem),
    ]
    d_cp = pltpu.async_copy(rcd_hbm.at[2, sid], depth_v, d_sem)
    for cp in rc_cps:
        cp.wait()

    boff = sid * (H // 8)
    copies = []
    for j in range(NCH):
        for k8 in range(CH // LANES):
            k = j * (CH // LANES) + k8
            r = rows_v[pl.ds(k * LANES, LANES)].astype(jnp.int32)
            c = cols_v[pl.ds(k * LANES, LANES)].astype(jnp.int32)
            tile = (boff + (r >> 3)) * 3 + (c >> 7)
            idx_v[pl.ds(k * LANES, LANES)] = (
                (tile << 10) + ((r & 7) << 7) + (c & 127))
        copies.append(pltpu.async_copy(
            img_hbm.at[idx_v.at[pl.ds(j * CH, CH)]],
            vals_v.at[pl.ds(j * CH, CH)], sem))
    d_cp.wait()

    acc = jnp.zeros((LANES,), jnp.float32)
    cnt = jnp.zeros((LANES,), jnp.float32)
    for j in range(NCH):
        copies[j].wait()
        for k8 in range(CH // LANES):
            k = j * (CH // LANES) + k8
            v = vals_v[pl.ds(k * LANES, LANES)]
            d = depth_v[pl.ds(k * LANES, LANES)]
            m = d > 0.0
            acc = acc + jnp.where(m, jnp.abs(v - d), 0.0)
            cnt = cnt + jnp.where(m, 1.0, 0.0)
    part_v[0, :] = acc
    part_v[1, :] = cnt
    pltpu.sync_copy(part_v, out_hbm.at[sid])


def _finalize_body(p_ref, o_ref):
    p = p_ref[...]
    s = jnp.sum(p[:, 0, :])
    c = jnp.sum(p[:, 1, :])
    loss = jnp.where(c > 0.0, s / jnp.maximum(c, 1.0), 0.0)
    o_ref[...] = jnp.broadcast_to(loss, (1, 1))


_finalize = pl.pallas_call(
    _finalize_body,
    out_shape=jax.ShapeDtypeStruct((1, 1), jnp.float32),
)


def kernel(output, rdepth):
    img = (output.reshape(B, H // 8, 8, W // 128, 128)
           .transpose(0, 1, 3, 2, 4)
           .reshape(-1))
    rcd = jnp.transpose(rdepth, (2, 0, 1))
    partials = _depth_partials(img, rcd)
    return _finalize(partials)[0, 0]


# speedup vs baseline: 1.0652x; 1.0272x over previous
import functools

import jax
import jax.numpy as jnp
from jax import lax
from jax.experimental import pallas as pl
from jax.experimental.pallas import tpu as pltpu
from jax.experimental.pallas import tpu_sc as plsc

B, H, W = 16, 384, 384
P = 512
NS = 16
PPT = (B * P) // NS
CH = 128
NCH = PPT // CH
LANES = 16

_mesh = plsc.VectorSubcoreMesh(
    core_axis_name="c", subcore_axis_name="s", num_cores=1)


def _lane_total(x):
    ids = lax.iota(jnp.int32, LANES)
    dnums = lax.GatherDimensionNumbers(
        offset_dims=(), collapsed_slice_dims=(0,), start_index_map=(0,))
    for shift in (1, 2, 4, 8):
        perm = (ids ^ shift).reshape(LANES, 1)
        x = x + lax.gather(x, perm, dnums, slice_sizes=(1,),
                           mode=lax.GatherScatterMode.PROMISE_IN_BOUNDS)
    return x


@functools.partial(
    pl.kernel,
    out_type=[jax.ShapeDtypeStruct((LANES,), jnp.float32),
              jax.ShapeDtypeStruct((NS, 2, LANES), jnp.float32)],
    mesh=_mesh,
    scratch_types=[
        pltpu.VMEM((PPT,), jnp.float32),
        pltpu.VMEM((PPT,), jnp.float32),
        pltpu.VMEM((PPT,), jnp.float32),
        pltpu.VMEM((PPT,), jnp.int32),
        pltpu.VMEM((PPT,), jnp.float32),
        pltpu.VMEM((2, LANES), jnp.float32),
        pltpu.VMEM((NS, 2, LANES), jnp.float32),
        pltpu.VMEM((LANES,), jnp.float32),
        pltpu.SemaphoreType.DMA,
        pltpu.SemaphoreType.DMA,
        pltpu.SemaphoreType.DMA,
    ],
)
def _depth_loss(img_hbm, rcd_hbm, out_hbm, parts_hbm,
                rows_v, cols_v, depth_v, idx_v, vals_v, part_v, all_v,
                out_v, rc_sem, d_sem, sem):
    sid = lax.axis_index("s")
    rc_cps = [
        pltpu.async_copy(rcd_hbm.at[0, sid], rows_v, rc_sem),
        pltpu.async_copy(rcd_hbm.at[1, sid], cols_v, rc_sem),
    ]
    d_cp = pltpu.async_copy(rcd_hbm.at[2, sid], depth_v, d_sem)
    for cp in rc_cps:
        cp.wait()

    boff = sid * (H // 8)
    copies = []
    for j in range(NCH):
        for k8 in range(CH // LANES):
            k = j * (CH // LANES) + k8
            r = rows_v[pl.ds(k * LANES, LANES)].astype(jnp.int32)
            c = cols_v[pl.ds(k * LANES, LANES)].astype(jnp.int32)
            tile = (boff + (r >> 3)) * 3 + (c >> 7)
            idx_v[pl.ds(k * LANES, LANES)] = (
                (tile << 10) + ((r & 7) << 7) + (c & 127))
        copies.append(pltpu.async_copy(
            img_hbm.at[idx_v.at[pl.ds(j * CH, CH)]],
            vals_v.at[pl.ds(j * CH, CH)], sem))
    d_cp.wait()

    acc = jnp.zeros((LANES,), jnp.float32)
    cnt = jnp.zeros((LANES,), jnp.float32)
    for j in range(NCH):
        copies[j].wait()
        for k8 in range(CH // LANES):
            k = j * (CH // LANES) + k8
            v = vals_v[pl.ds(k * LANES, LANES)]
            d = depth_v[pl.ds(k * LANES, LANES)]
            m = d > 0.0
            acc = acc + jnp.where(m, jnp.abs(v - d), 0.0)
            cnt = cnt + jnp.where(m, 1.0, 0.0)
    part_v[0, :] = acc
    part_v[1, :] = cnt
    pltpu.sync_copy(part_v, parts_hbm.at[sid])
    plsc.subcore_barrier()

    @pl.when(sid == 0)
    def _finalize():
        pltpu.sync_copy(parts_hbm, all_v)
        s = jnp.zeros((LANES,), jnp.float32)
        c = jnp.zeros((LANES,), jnp.float32)
        for i in range(NS):
            s = s + all_v[i, 0, :]
            c = c + all_v[i, 1, :]
        st = _lane_total(s)
        ct = _lane_total(c)
        out_v[...] = jnp.where(
            ct > 0.0, st / jnp.maximum(ct, 1.0),
            jnp.zeros((LANES,), jnp.float32))
        pltpu.sync_copy(out_v, out_hbm)


def kernel(output, rdepth):
    img = (output.reshape(B, H // 8, 8, W // 128, 128)
           .transpose(0, 1, 3, 2, 4)
           .reshape(-1))
    rcd = jnp.transpose(rdepth, (2, 0, 1))
    loss, _ = _depth_loss(img, rcd)
    return loss[0]


# speedup vs baseline: 1.0846x; 1.0182x over previous
import functools

import jax
import jax.numpy as jnp
from jax import lax
from jax.experimental import pallas as pl
from jax.experimental.pallas import tpu as pltpu
from jax.experimental.pallas import tpu_sc as plsc

B, H, W = 16, 384, 384
P = 512
NS = 16
PPT = (B * P) // NS
CH = 128
NCH = PPT // CH
LANES = 16

_mesh = plsc.VectorSubcoreMesh(
    core_axis_name="c", subcore_axis_name="s", num_cores=1)


def _lane_total(x):
    ids = lax.iota(jnp.int32, LANES)
    dnums = lax.GatherDimensionNumbers(
        offset_dims=(), collapsed_slice_dims=(0,), start_index_map=(0,))
    for shift in (1, 2, 4, 8):
        perm = (ids ^ shift).reshape(LANES, 1)
        x = x + lax.gather(x, perm, dnums, slice_sizes=(1,),
                           mode=lax.GatherScatterMode.PROMISE_IN_BOUNDS)
    return x


@functools.partial(
    pl.kernel,
    out_type=[jax.ShapeDtypeStruct((LANES,), jnp.float32),
              jax.ShapeDtypeStruct((NS, 2, LANES), jnp.float32)],
    mesh=_mesh,
    scratch_types=[
        pltpu.VMEM((PPT,), jnp.float32),
        pltpu.VMEM((PPT,), jnp.float32),
        pltpu.VMEM((PPT,), jnp.float32),
        pltpu.VMEM((PPT,), jnp.int32),
        pltpu.VMEM((PPT,), jnp.float32),
        pltpu.VMEM((2, LANES), jnp.float32),
        pltpu.VMEM((NS, 2, LANES), jnp.float32),
        pltpu.VMEM((LANES,), jnp.float32),
        pltpu.SemaphoreType.DMA,
        pltpu.SemaphoreType.DMA,
        pltpu.SemaphoreType.DMA,
    ],
)
def _depth_loss(img_hbm, rcd_hbm, out_hbm, parts_hbm,
                rows_v, cols_v, depth_v, idx_v, vals_v, part_v, all_v,
                out_v, rc_sem, d_sem, sem):
    sid = lax.axis_index("s")
    rc_cps = [
        pltpu.async_copy(rcd_hbm.at[0, sid], rows_v, rc_sem),
        pltpu.async_copy(rcd_hbm.at[1, sid], cols_v, rc_sem),
    ]
    d_cp = pltpu.async_copy(rcd_hbm.at[2, sid], depth_v, d_sem)
    for cp in rc_cps:
        cp.wait()

    boff = sid * (H // 8)

    def _idx_body(k, carry):
        r = rows_v[pl.ds(k * LANES, LANES)].astype(jnp.int32)
        c = cols_v[pl.ds(k * LANES, LANES)].astype(jnp.int32)
        tile = (boff + (r >> 3)) * 3 + (c >> 7)
        idx_v[pl.ds(k * LANES, LANES)] = (
            (tile << 10) + ((r & 7) << 7) + (c & 127))
        return carry

    lax.fori_loop(0, PPT // LANES, _idx_body, 0)
    copies = []
    for j in range(NCH):
        copies.append(pltpu.async_copy(
            img_hbm.at[idx_v.at[pl.ds(j * CH, CH)]],
            vals_v.at[pl.ds(j * CH, CH)], sem))
    d_cp.wait()
    for cp in copies:
        cp.wait()

    def _acc_body(k, carry):
        acc, cnt = carry
        v = vals_v[pl.ds(k * LANES, LANES)]
        d = depth_v[pl.ds(k * LANES, LANES)]
        m = d > 0.0
        acc = acc + jnp.where(m, jnp.abs(v - d), 0.0)
        cnt = cnt + jnp.where(m, 1.0, 0.0)
        return acc, cnt

    acc, cnt = lax.fori_loop(
        0, PPT // LANES, _acc_body,
        (jnp.zeros((LANES,), jnp.float32), jnp.zeros((LANES,), jnp.float32)))
    part_v[0, :] = acc
    part_v[1, :] = cnt
    pltpu.sync_copy(part_v, parts_hbm.at[sid])
    plsc.subcore_barrier()

    @pl.when(sid == 0)
    def _finalize():
        pltpu.sync_copy(parts_hbm, all_v)
        s = jnp.zeros((LANES,), jnp.float32)
        c = jnp.zeros((LANES,), jnp.float32)
        for i in range(NS):
            s = s + all_v[i, 0, :]
            c = c + all_v[i, 1, :]
        st = _lane_total(s)
        ct = _lane_total(c)
        out_v[...] = jnp.where(
            ct > 0.0, st / jnp.maximum(ct, 1.0),
            jnp.zeros((LANES,), jnp.float32))
        pltpu.sync_copy(out_v, out_hbm)


def kernel(output, rdepth):
    img = (output.reshape(B, H // 8, 8, W // 128, 128)
           .transpose(0, 1, 3, 2, 4)
           .reshape(-1))
    rcd = jnp.transpose(rdepth, (2, 0, 1))
    loss, _ = _depth_loss(img, rcd)
    return loss[0]


# speedup vs baseline: 1.0904x; 1.0054x over previous
import functools

import jax
import jax.numpy as jnp
from jax import lax
from jax.experimental import pallas as pl
from jax.experimental.pallas import tpu as pltpu
from jax.experimental.pallas import tpu_sc as plsc

B, H, W = 16, 384, 384
P = 512
NS = 16
PPT = (B * P) // NS
CH = 128
NCH = PPT // CH
LANES = 16

_mesh = plsc.VectorSubcoreMesh(
    core_axis_name="c", subcore_axis_name="s", num_cores=1)


def _lane_total(x):
    ids = lax.iota(jnp.int32, LANES)
    dnums = lax.GatherDimensionNumbers(
        offset_dims=(), collapsed_slice_dims=(0,), start_index_map=(0,))
    for shift in (1, 2, 4, 8):
        perm = (ids ^ shift).reshape(LANES, 1)
        x = x + lax.gather(x, perm, dnums, slice_sizes=(1,),
                           mode=lax.GatherScatterMode.PROMISE_IN_BOUNDS)
    return x


@functools.partial(
    pl.kernel,
    out_type=[jax.ShapeDtypeStruct((LANES,), jnp.float32),
              jax.ShapeDtypeStruct((NS, 2, LANES), jnp.float32)],
    mesh=_mesh,
    scratch_types=[
        pltpu.VMEM((PPT,), jnp.float32),
        pltpu.VMEM((PPT,), jnp.float32),
        pltpu.VMEM((PPT,), jnp.float32),
        pltpu.VMEM((PPT,), jnp.int32),
        pltpu.VMEM((PPT,), jnp.float32),
        pltpu.VMEM((2, LANES), jnp.float32),
        pltpu.VMEM((NS, 2, LANES), jnp.float32),
        pltpu.VMEM((LANES,), jnp.float32),
        pltpu.SemaphoreType.DMA,
        pltpu.SemaphoreType.DMA,
        pltpu.SemaphoreType.DMA,
    ],
)
def _depth_loss(img_hbm, rcd_hbm, out_hbm, parts_hbm,
                rows_v, cols_v, depth_v, idx_v, vals_v, part_v, all_v,
                out_v, rc_sem, d_sem, sem):
    sid = lax.axis_index("s")
    rc_cps = [
        pltpu.async_copy(rcd_hbm.at[0, sid], rows_v, rc_sem),
        pltpu.async_copy(rcd_hbm.at[1, sid], cols_v, rc_sem),
    ]
    d_cp = pltpu.async_copy(rcd_hbm.at[2, sid], depth_v, d_sem)
    for cp in rc_cps:
        cp.wait()

    boff = sid * (H // 8)

    def _idx_body(k, carry):
        r = rows_v[pl.ds(k * LANES, LANES)].astype(jnp.int32)
        c = cols_v[pl.ds(k * LANES, LANES)].astype(jnp.int32)
        tile = (boff + (r >> 3)) * 3 + (c >> 7)
        idx_v[pl.ds(k * LANES, LANES)] = (
            (tile << 10) + ((r & 7) << 7) + (c & 127))
        return carry

    lax.fori_loop(0, PPT // LANES, _idx_body, 0)
    copies = []
    for j in range(NCH):
        copies.append(pltpu.async_copy(
            img_hbm.at[idx_v.at[pl.ds(j * CH, CH)]],
            vals_v.at[pl.ds(j * CH, CH)], sem))
    d_cp.wait()
    for cp in copies:
        cp.wait()

    def _acc_body(k, carry):
        acc, cnt = carry
        v = vals_v[pl.ds(k * LANES, LANES)]
        d = depth_v[pl.ds(k * LANES, LANES)]
        m = d > 0.0
        acc = acc + jnp.where(m, jnp.abs(v - d), 0.0)
        cnt = cnt + jnp.where(m, 1.0, 0.0)
        return acc, cnt

    acc, cnt = lax.fori_loop(
        0, PPT // LANES, _acc_body,
        (jnp.zeros((LANES,), jnp.float32), jnp.zeros((LANES,), jnp.float32)))
    part_v[0, :] = acc
    part_v[1, :] = cnt
    pltpu.sync_copy(part_v, parts_hbm.at[sid])
    plsc.subcore_barrier()

    @pl.when(sid == 0)
    def _finalize():
        pltpu.sync_copy(parts_hbm, all_v)

        def _red_body(i, carry):
            s, c = carry
            return s + all_v[i, 0, :], c + all_v[i, 1, :]

        s, c = lax.fori_loop(
            0, NS, _red_body,
            (jnp.zeros((LANES,), jnp.float32),
             jnp.zeros((LANES,), jnp.float32)))
        st = _lane_total(s)
        ct = _lane_total(c)
        out_v[...] = jnp.where(
            ct > 0.0, st / jnp.maximum(ct, 1.0),
            jnp.zeros((LANES,), jnp.float32))
        pltpu.sync_copy(out_v, out_hbm)


def kernel(output, rdepth):
    img = (output.reshape(B, H // 8, 8, W // 128, 128)
           .transpose(0, 1, 3, 2, 4)
           .reshape(-1))
    rcd = jnp.transpose(rdepth, (2, 0, 1))
    loss, _ = _depth_loss(img, rcd)
    return loss[0]
